# Initial kernel scaffold; baseline (speedup 1.0000x reference)
import jax
import jax.numpy as jnp
from jax.experimental import pallas as pl


def kernel(indices, tt_core):
    raise NotImplementedError("write your pallas kernel here")


# speedup vs baseline: 3.9733x; 3.9733x over previous
import functools

import jax
import jax.numpy as jnp
from jax import lax
from jax.experimental import pallas as pl
from jax.experimental.pallas import tpu as pltpu
from jax.experimental.pallas import tpu_sc as plsc

R1 = 32
R2 = 32
N1 = 200
N2 = 200
B = 16384

NW = 32
CHUNK = 32
ROWS = CHUNK * R1
GL = 128
NG = ROWS // GL
PER_W = B // NW
NCH = PER_W // CHUNK


def _sc_gather(ind_flat, table):
    mesh = plsc.VectorSubcoreMesh(core_axis_name="c", subcore_axis_name="s")

    @functools.partial(
        pl.kernel,
        mesh=mesh,
        out_type=jax.ShapeDtypeStruct((B * R1, R2), jnp.float32),
        compiler_params=pltpu.CompilerParams(use_tc_tiling_on_sc=False),
        scratch_types=[
            pltpu.VMEM((CHUNK,), jnp.int32),
            pltpu.VMEM((CHUNK,), jnp.int32),
            pltpu.VMEM((ROWS,), jnp.int32),
            pltpu.VMEM((ROWS, R2), jnp.float32),
            pltpu.SemaphoreType.DMA,
        ],
    )
    def k(ind_hbm, tab_hbm, out_hbm, i0_v, i1_v, idx_v, rows_v, sem):
        wid = lax.axis_index("s") * 2 + lax.axis_index("c")
        lane = lax.iota(jnp.int32, 16)
        ramp_lo = lane * (N1 * N2)
        ramp_hi = ramp_lo + 16 * (N1 * N2)

        def chunk_body(g, carry):
            cbase = wid * PER_W + g * CHUNK
            pltpu.sync_copy(ind_hbm.at[pl.ds(cbase, CHUNK)], i0_v)
            pltpu.sync_copy(ind_hbm.at[pl.ds(B + cbase, CHUNK)], i1_v)
            for c_off in (0, 16):
                jvec = i0_v[pl.ds(c_off, 16)] * N2 + i1_v[pl.ds(c_off, 16)]
                for t in range(16):
                    j = jvec[t]
                    c = c_off + t
                    idx_v[pl.ds(c * R1, 16)] = ramp_lo + j
                    idx_v[pl.ds(c * R1 + 16, 16)] = ramp_hi + j
            copies = [
                pltpu.async_copy(
                    tab_hbm.at[idx_v.at[pl.ds(i * GL, GL)]],
                    rows_v.at[pl.ds(i * GL, GL), :],
                    sem,
                )
                for i in range(NG)
            ]
            for c in copies:
                c.wait()
            pltpu.sync_copy(rows_v, out_hbm.at[pl.ds(cbase * R1, ROWS), :])
            return carry

        lax.fori_loop(0, NCH, chunk_body, 0)

    return k(ind_flat, table)


def kernel(indices, tt_core):
    ind_flat = indices.T.reshape(-1)
    table = tt_core.reshape(R1 * N1 * N2, R2)
    out = _sc_gather(ind_flat, table)
    return out.reshape(B, R1, R2)


# speedup vs baseline: 4.0644x; 1.0229x over previous
import functools

import jax
import jax.numpy as jnp
from jax import lax
from jax.experimental import pallas as pl
from jax.experimental.pallas import tpu as pltpu
from jax.experimental.pallas import tpu_sc as plsc

R1 = 32
R2 = 32
N1 = 200
N2 = 200
B = 16384

NW = 32
CHUNK = 32
ROWS = CHUNK * R1
GL = 128
NG = ROWS // GL
PER_W = B // NW
NCH = PER_W // CHUNK
NBUF = 3


def _sc_gather(ind_flat, table):
    mesh = plsc.VectorSubcoreMesh(core_axis_name="c", subcore_axis_name="s")

    @functools.partial(
        pl.kernel,
        mesh=mesh,
        out_type=jax.ShapeDtypeStruct((B * R1, R2), jnp.float32),
        compiler_params=pltpu.CompilerParams(use_tc_tiling_on_sc=False),
        scratch_types=[
            pltpu.VMEM((PER_W,), jnp.int32),
            pltpu.VMEM((PER_W,), jnp.int32),
            pltpu.VMEM((NBUF, ROWS), jnp.int32),
            pltpu.VMEM((NBUF, ROWS, R2), jnp.float32),
            pltpu.SemaphoreType.DMA,
            pltpu.SemaphoreType.DMA,
            pltpu.SemaphoreType.DMA,
            pltpu.SemaphoreType.DMA,
        ],
    )
    def k(ind_hbm, tab_hbm, out_hbm, i0_v, i1_v, idx_v, rows_v,
          gsem0, gsem1, osem0, osem1):
        wid = lax.axis_index("s") * 2 + lax.axis_index("c")
        base = wid * PER_W
        lane = lax.iota(jnp.int32, 16)
        ramp_lo = lane * (N1 * N2)
        ramp_hi = ramp_lo + 16 * (N1 * N2)
        gsems = (gsem0, gsem1)
        osems = (osem0, osem1)

        pltpu.sync_copy(ind_hbm.at[pl.ds(base, PER_W)], i0_v)
        pltpu.sync_copy(ind_hbm.at[pl.ds(B + base, PER_W)], i1_v)

        gather_h = [None] * NCH
        out_h = [None] * NCH

        def build_idx(g):
            idx_s = idx_v.at[g % NBUF]
            for c_off in (0, 16):
                cs = g * CHUNK + c_off
                jvec = i0_v[pl.ds(cs, 16)] * N2 + i1_v[pl.ds(cs, 16)]
                for t in range(16):
                    j = jvec[t]
                    p = (c_off + t) * R1
                    idx_s[pl.ds(p, 16)] = ramp_lo + j
                    idx_s[pl.ds(p + 16, 16)] = ramp_hi + j

        def fire_gathers(g):
            s = g % NBUF
            gather_h[g] = [
                pltpu.async_copy(
                    tab_hbm.at[idx_v.at[s].at[pl.ds(i * GL, GL)]],
                    rows_v.at[s].at[pl.ds(i * GL, GL), :],
                    gsems[g % 2],
                )
                for i in range(NG)
            ]

        def fire_out(g):
            s = g % NBUF
            out_h[g] = pltpu.async_copy(
                rows_v.at[s],
                out_hbm.at[pl.ds((base + g * CHUNK) * R1, ROWS), :],
                osems[g % 2],
            )

        for g in range(NCH):
            build_idx(g)
            if g >= NBUF:
                out_h[g - NBUF].wait()
            fire_gathers(g)
            if g >= 1:
                for h in gather_h[g - 1]:
                    h.wait()
                fire_out(g - 1)
        for h in gather_h[NCH - 1]:
            h.wait()
        fire_out(NCH - 1)
        for g in range(NCH - NBUF + 1, NCH):
            out_h[g].wait()

    return k(ind_flat, table)


def kernel(indices, tt_core):
    ind_flat = indices.T.reshape(-1)
    table = tt_core.reshape(R1 * N1 * N2, R2)
    out = _sc_gather(ind_flat, table)
    return out.reshape(B, R1, R2)
